# baseline (speedup 1.0000x reference)
import dataclasses

import jax
import jax.numpy as jnp
from jax import lax
from jax.experimental import pallas as pl
from jax.experimental.pallas import tpu as pltpu
from jax.experimental.pallas import tpu_sc as plsc

N = 10000
D = 128
E = 320000
FC = 256
NC = 8

NPAD = 10240
EPAD = 327680
NSUB = 16
NCORE = 2
NW = NSUB * NCORE
ROWS_PER_SUB = NPAD // NSUB
CHUNK = 128
CHUNKS = EPAD // (NW * CHUNK)
C0 = 38
C1 = 122
EROWS = EPAD // CHUNK
BLK = 512

_SC_CP = pltpu.CompilerParams()
if "needs_layout_passes" in pltpu.CompilerParams.__dataclass_fields__:
    _SC_CP = dataclasses.replace(_SC_CP, needs_layout_passes=False)


def _seg_sum_sc(values, src2d, dst2d, zero_d, with_counts):
    mesh = plsc.VectorSubcoreMesh(core_axis_name="c", subcore_axis_name="s")
    out_type = [jax.ShapeDtypeStruct((NCORE, NPAD, D), jnp.float32)]
    scratch = [
        pltpu.VMEM((CHUNK,), jnp.int32),
        pltpu.VMEM((CHUNK,), jnp.int32),
        pltpu.VMEM((CHUNK,), jnp.int32),
        pltpu.VMEM((CHUNK,), jnp.int32),
        pltpu.VMEM((CHUNK, D), jnp.float32),
        pltpu.VMEM((CHUNK, D), jnp.float32),
        pltpu.SemaphoreType.DMA,
        pltpu.SemaphoreType.DMA,
        pltpu.VMEM_SHARED((NPAD, D), jnp.float32),
    ]
    if with_counts:
        out_type.append(jax.ShapeDtypeStruct((NW, NPAD), jnp.float32))
        scratch.append(pltpu.VMEM((NPAD,), jnp.float32))

    def body(x_hbm, s_hbm, d_hbm, z_hbm, p_hbm, c_hbm,
             idx_s0, idx_d0, idx_s1, idx_d1, rows0, rows1, sem0, sem1,
             acc, lcnt):
        cid = lax.axis_index("c")
        sid = lax.axis_index("s")
        gw = cid * NSUB + sid
        r0 = sid * ROWS_PER_SUB

        @pl.loop(0, ROWS_PER_SUB // CHUNK)
        def _(i):
            pltpu.sync_copy(z_hbm.at[pl.ds(r0 + i * CHUNK, CHUNK)], rows0)
            pltpu.sync_copy(rows0, acc.at[pl.ds(r0 + i * CHUNK, CHUNK)])
        if with_counts:
            @pl.loop(0, NPAD, step=16)
            def _(i):
                lcnt[pl.ds(i, 16)] = jnp.zeros((16,), jnp.float32)
        plsc.subcore_barrier()

        ones16 = jnp.ones((16,), jnp.float32)

        def hist(idx_ref):
            @pl.loop(0, CHUNK, step=16)
            def _(k2):
                idx16 = idx_ref[pl.ds(k2, 16)]
                plsc.addupdate_scatter(lcnt, [idx16], ones16)

        nchunks = jnp.where(cid == 0, C0, C1)
        base = jnp.where(cid == 0, sid * C0, NSUB * C0 + sid * C1)

        @pl.loop(0, nchunks, step=2)
        def _(j):
            row = base + j
            pltpu.sync_copy(s_hbm.at[row], idx_s0)
            pltpu.sync_copy(d_hbm.at[row], idx_d0)
            cp0 = pltpu.async_copy(x_hbm.at[idx_s0], rows0, sem0)
            pltpu.sync_copy(s_hbm.at[row + 1], idx_s1)
            pltpu.sync_copy(d_hbm.at[row + 1], idx_d1)
            cp1 = pltpu.async_copy(x_hbm.at[idx_s1], rows1, sem1)
            if with_counts:
                hist(idx_d0)
            cp0.wait()
            pltpu.sync_copy(rows0, acc.at[idx_d0], add=True)
            if with_counts:
                hist(idx_d1)
            cp1.wait()
            pltpu.sync_copy(rows1, acc.at[idx_d1], add=True)

        plsc.subcore_barrier()

        @pl.loop(0, ROWS_PER_SUB // CHUNK)
        def _(i):
            pltpu.sync_copy(acc.at[pl.ds(r0 + i * CHUNK, CHUNK)], rows0)
            pltpu.sync_copy(rows0, p_hbm.at[cid, pl.ds(r0 + i * CHUNK, CHUNK)])
        if with_counts:
            pltpu.sync_copy(lcnt, c_hbm.at[gw])

    if with_counts:
        def body_wc(x, s, d, z, p, c, s0, d0, s1, d1, r_0, r_1, m0, m1,
                    acc, lcnt):
            body(x, s, d, z, p, c, s0, d0, s1, d1, r_0, r_1, m0, m1,
                 acc, lcnt)
        k = pl.kernel(body_wc, out_type=out_type, mesh=mesh,
                      compiler_params=_SC_CP, scratch_types=scratch)
    else:
        def body_nc(x, s, d, z, p, s0, d0, s1, d1, r_0, r_1, m0, m1, acc):
            body(x, s, d, z, p, None, s0, d0, s1, d1, r_0, r_1, m0, m1,
                 acc, None)
        k = pl.kernel(body_nc, out_type=out_type, mesh=mesh,
                      scratch_types=scratch)
    return k(values, src2d, dst2d, zero_d)


def _mm(a, b):
    return jnp.dot(a, b, preferred_element_type=jnp.float32,
                   precision=lax.Precision.HIGHEST)


def _tc1_body(x_ref, w_ref, b_ref, o_ref):
    o_ref[...] = _mm(x_ref[...], w_ref[...]) + b_ref[...]


def _tc2_body(p0_ref, p1_ref, c_ref, r1_ref, w1lT_ref, w2rT_ref,
              b2l_ref, h_ref, r2_ref):
    s = p0_ref[0] + p1_ref[0]
    cntv = jnp.maximum(jnp.sum(c_ref[...], axis=0), 1.0)[:, None]
    mean = s / cntv
    o = _mm(mean, w1lT_ref[...]) + r1_ref[...]
    nrm = jnp.sqrt(jnp.sum(o * o, axis=-1, keepdims=True))
    h = jnp.maximum(o / jnp.maximum(nrm, 1e-12), 0.0)
    h_ref[...] = h
    r2_ref[...] = _mm(h, w2rT_ref[...]) + b2l_ref[...]


def _tc3_body(p0_ref, p1_ref, c_ref, r2_ref, w2lT_ref, wc1T_ref,
              bc1_ref, wc2T_ref, bc2_ref, h2_ref, out_ref):
    s = p0_ref[0] + p1_ref[0]
    cntv = jnp.maximum(jnp.sum(c_ref[...], axis=0), 1.0)[:, None]
    mean = s / cntv
    o = _mm(mean, w2lT_ref[...]) + r2_ref[...]
    nrm = jnp.sqrt(jnp.sum(o * o, axis=-1, keepdims=True))
    h2 = o / jnp.maximum(nrm, 1e-12)
    h2_ref[...] = h2
    fc = jnp.maximum(_mm(h2, wc1T_ref[...]) + bc1_ref[...], 0.0)
    out_ref[...] = _mm(fc, wc2T_ref[...]) + bc2_ref[...]


def _rows_spec(last):
    return pl.BlockSpec((BLK, last), lambda i: (i, 0))


def _part_spec(last, core):
    return pl.BlockSpec((1, BLK, last), lambda i, c=core: (c, i, 0))


def _cnt_spec():
    return pl.BlockSpec((NW, BLK), lambda i: (0, i))


def _full_spec(r, c):
    return pl.BlockSpec((r, c), lambda i: (0, 0))


def _tc1(xp, w1rT, b1l2d):
    return pl.pallas_call(
        _tc1_body,
        grid=(NPAD // BLK,),
        in_specs=[_rows_spec(D), _full_spec(D, D), _full_spec(1, D)],
        out_specs=_rows_spec(D),
        out_shape=jax.ShapeDtypeStruct((NPAD, D), jnp.float32),
    )(xp, w1rT, b1l2d)


def _tc2(p, c, r1, w1lT, w2rT, b2l2d):
    return pl.pallas_call(
        _tc2_body,
        grid=(NPAD // BLK,),
        in_specs=[_part_spec(D, 0), _part_spec(D, 1), _cnt_spec(),
                  _rows_spec(D), _full_spec(D, D), _full_spec(D, D),
                  _full_spec(1, D)],
        out_specs=[_rows_spec(D), _rows_spec(D)],
        out_shape=[jax.ShapeDtypeStruct((NPAD, D), jnp.float32),
                   jax.ShapeDtypeStruct((NPAD, D), jnp.float32)],
    )(p, p, c, r1, w1lT, w2rT, b2l2d)


def _tc3(p, c, r2, w2lT, wc1T, bc12d, wc2Tp, bc2p):
    return pl.pallas_call(
        _tc3_body,
        grid=(NPAD // BLK,),
        in_specs=[_part_spec(D, 0), _part_spec(D, 1), _cnt_spec(),
                  _rows_spec(D), _full_spec(D, D), _full_spec(D, FC),
                  _full_spec(1, FC), _full_spec(FC, D), _full_spec(1, D)],
        out_specs=[_rows_spec(D), _rows_spec(D)],
        out_shape=[jax.ShapeDtypeStruct((NPAD, D), jnp.float32),
                   jax.ShapeDtypeStruct((NPAD, D), jnp.float32)],
    )(p, p, c, r2, w2lT, wc1T, bc12d, wc2Tp, bc2p)


def kernel(x, edge_index, W1l, b1l, W1r, W2l, b2l, W2r, Wc1, bc1, Wc2, bc2):
    xp = jnp.pad(x, ((0, NPAD - N), (0, 0)))
    src = edge_index[0]
    dst = edge_index[1]
    srcp = jnp.pad(src, (0, EPAD - E)).reshape(EROWS, CHUNK)
    dstp = jnp.pad(dst, (0, EPAD - E), constant_values=N).reshape(EROWS, CHUNK)
    zero_d = jnp.zeros((NPAD, D), jnp.float32)

    p1, cnt32 = _seg_sum_sc(xp, srcp, dstp, zero_d, True)
    r1 = _tc1(xp, W1r.T, b1l[None, :])
    h, r2 = _tc2(p1, cnt32, r1, W1l.T, W2r.T, b2l[None, :])
    (p2,) = _seg_sum_sc(h, srcp, dstp, zero_d, False)
    wc2Tp = jnp.pad(Wc2.T, ((0, 0), (0, D - NC)))
    bc2p = jnp.pad(bc2, (0, D - NC))[None, :]
    h2p, outp = _tc3(p2, cnt32, r2, W2l.T, Wc1.T, bc1[None, :], wc2Tp, bc2p)

    out = outp[:N, :NC]
    h2 = h2p[:N]
    node_mask = (jax.random.uniform(jax.random.key(1), (N, 1)) > 0.1)
    return (out, node_mask.astype(jnp.float32), h2)


# speedup vs baseline: 1.1203x; 1.1203x over previous
import dataclasses

import jax
import jax.numpy as jnp
from jax import lax
from jax.experimental import pallas as pl
from jax.experimental.pallas import tpu as pltpu
from jax.experimental.pallas import tpu_sc as plsc

N = 10000
D = 128
E = 320000
FC = 256
NC = 8

NPAD = 10240
EPAD = 327680
NSUB = 16
NCORE = 2
NW = NSUB * NCORE
ROWS_PER_SUB = NPAD // NSUB
CHUNK = 128
CHUNKS = EPAD // (NW * CHUNK)
C0 = 80
C1 = 80
EROWS = EPAD // CHUNK
BLK = 512

_SC_CP = pltpu.CompilerParams()
if "needs_layout_passes" in pltpu.CompilerParams.__dataclass_fields__:
    _SC_CP = dataclasses.replace(_SC_CP, needs_layout_passes=False)


def _seg_sum_sc(values, src2d, dst2d, zero_d, with_counts):
    mesh = plsc.VectorSubcoreMesh(core_axis_name="c", subcore_axis_name="s")
    out_type = [jax.ShapeDtypeStruct((NCORE, NPAD, D), jnp.float32)]
    scratch = [
        pltpu.VMEM((CHUNK,), jnp.int32),
        pltpu.VMEM((CHUNK,), jnp.int32),
        pltpu.VMEM((CHUNK,), jnp.int32),
        pltpu.VMEM((CHUNK,), jnp.int32),
        pltpu.VMEM((CHUNK, D), jnp.float32),
        pltpu.VMEM((CHUNK, D), jnp.float32),
        pltpu.SemaphoreType.DMA,
        pltpu.SemaphoreType.DMA,
        pltpu.VMEM_SHARED((NPAD, D), jnp.float32),
    ]
    if with_counts:
        out_type.append(jax.ShapeDtypeStruct((NW, NPAD), jnp.float32))
        scratch.append(pltpu.VMEM((NPAD,), jnp.float32))

    def body(x_hbm, s_hbm, d_hbm, z_hbm, p_hbm, c_hbm,
             idx_s0, idx_d0, idx_s1, idx_d1, rows0, rows1, sem0, sem1,
             acc, lcnt):
        cid = lax.axis_index("c")
        sid = lax.axis_index("s")
        gw = cid * NSUB + sid
        r0 = sid * ROWS_PER_SUB

        pltpu.sync_copy(z_hbm.at[pl.ds(r0, ROWS_PER_SUB)],
                        acc.at[pl.ds(r0, ROWS_PER_SUB)])
        if with_counts:
            @pl.loop(0, NPAD, step=16)
            def _(i):
                lcnt[pl.ds(i, 16)] = jnp.zeros((16,), jnp.float32)
        plsc.subcore_barrier()

        ones16 = jnp.ones((16,), jnp.float32)

        def hist(idx_ref):
            @pl.loop(0, CHUNK, step=16)
            def _(k2):
                idx16 = idx_ref[pl.ds(k2, 16)]
                plsc.addupdate_scatter(lcnt, [idx16], ones16)

        nchunks = jnp.where(cid == 0, C0, C1)
        base = jnp.where(cid == 0, sid * C0, NSUB * C0 + sid * C1)

        @pl.loop(0, nchunks, step=2)
        def _(j):
            row = base + j
            pltpu.sync_copy(s_hbm.at[row], idx_s0)
            pltpu.sync_copy(d_hbm.at[row], idx_d0)
            cp0 = pltpu.async_copy(x_hbm.at[idx_s0], rows0, sem0)
            pltpu.sync_copy(s_hbm.at[row + 1], idx_s1)
            pltpu.sync_copy(d_hbm.at[row + 1], idx_d1)
            cp1 = pltpu.async_copy(x_hbm.at[idx_s1], rows1, sem1)
            if with_counts:
                hist(idx_d0)
            cp0.wait()
            pltpu.sync_copy(rows0, acc.at[idx_d0], add=True)
            if with_counts:
                hist(idx_d1)
            cp1.wait()
            pltpu.sync_copy(rows1, acc.at[idx_d1], add=True)

        plsc.subcore_barrier()

        pltpu.sync_copy(acc.at[pl.ds(r0, ROWS_PER_SUB)],
                        p_hbm.at[cid, pl.ds(r0, ROWS_PER_SUB)])
        if with_counts:
            pltpu.sync_copy(lcnt, c_hbm.at[gw])

    if with_counts:
        def body_wc(x, s, d, z, p, c, s0, d0, s1, d1, r_0, r_1, m0, m1,
                    acc, lcnt):
            body(x, s, d, z, p, c, s0, d0, s1, d1, r_0, r_1, m0, m1,
                 acc, lcnt)
        k = pl.kernel(body_wc, out_type=out_type, mesh=mesh,
                      compiler_params=_SC_CP, scratch_types=scratch)
    else:
        def body_nc(x, s, d, z, p, s0, d0, s1, d1, r_0, r_1, m0, m1, acc):
            body(x, s, d, z, p, None, s0, d0, s1, d1, r_0, r_1, m0, m1,
                 acc, None)
        k = pl.kernel(body_nc, out_type=out_type, mesh=mesh,
                      scratch_types=scratch)
    return k(values, src2d, dst2d, zero_d)


def _mm(a, b):
    return jnp.dot(a, b, preferred_element_type=jnp.float32,
                   precision=lax.Precision.HIGHEST)


def _tc1_body(x_ref, w_ref, b_ref, o_ref):
    o_ref[...] = _mm(x_ref[...], w_ref[...]) + b_ref[...]


def _tc2_body(p0_ref, p1_ref, c_ref, r1_ref, w1lT_ref, w2rT_ref,
              b2l_ref, h_ref, r2_ref):
    s = p0_ref[0] + p1_ref[0]
    cntv = jnp.maximum(jnp.sum(c_ref[...], axis=0), 1.0)[:, None]
    mean = s / cntv
    o = _mm(mean, w1lT_ref[...]) + r1_ref[...]
    nrm = jnp.sqrt(jnp.sum(o * o, axis=-1, keepdims=True))
    h = jnp.maximum(o / jnp.maximum(nrm, 1e-12), 0.0)
    h_ref[...] = h
    r2_ref[...] = _mm(h, w2rT_ref[...]) + b2l_ref[...]


def _tc3_body(p0_ref, p1_ref, c_ref, r2_ref, w2lT_ref, wc1T_ref,
              bc1_ref, wc2T_ref, bc2_ref, h2_ref, out_ref):
    s = p0_ref[0] + p1_ref[0]
    cntv = jnp.maximum(jnp.sum(c_ref[...], axis=0), 1.0)[:, None]
    mean = s / cntv
    o = _mm(mean, w2lT_ref[...]) + r2_ref[...]
    nrm = jnp.sqrt(jnp.sum(o * o, axis=-1, keepdims=True))
    h2 = o / jnp.maximum(nrm, 1e-12)
    h2_ref[...] = h2
    fc = jnp.maximum(_mm(h2, wc1T_ref[...]) + bc1_ref[...], 0.0)
    out_ref[...] = _mm(fc, wc2T_ref[...]) + bc2_ref[...]


def _rows_spec(last):
    return pl.BlockSpec((BLK, last), lambda i: (i, 0))


def _part_spec(last, core):
    return pl.BlockSpec((1, BLK, last), lambda i, c=core: (c, i, 0))


def _cnt_spec():
    return pl.BlockSpec((NW, BLK), lambda i: (0, i))


def _full_spec(r, c):
    return pl.BlockSpec((r, c), lambda i: (0, 0))


def _tc1(xp, w1rT, b1l2d):
    return pl.pallas_call(
        _tc1_body,
        grid=(NPAD // BLK,),
        in_specs=[_rows_spec(D), _full_spec(D, D), _full_spec(1, D)],
        out_specs=_rows_spec(D),
        out_shape=jax.ShapeDtypeStruct((NPAD, D), jnp.float32),
    )(xp, w1rT, b1l2d)


def _tc2(p, c, r1, w1lT, w2rT, b2l2d):
    return pl.pallas_call(
        _tc2_body,
        grid=(NPAD // BLK,),
        in_specs=[_part_spec(D, 0), _part_spec(D, 1), _cnt_spec(),
                  _rows_spec(D), _full_spec(D, D), _full_spec(D, D),
                  _full_spec(1, D)],
        out_specs=[_rows_spec(D), _rows_spec(D)],
        out_shape=[jax.ShapeDtypeStruct((NPAD, D), jnp.float32),
                   jax.ShapeDtypeStruct((NPAD, D), jnp.float32)],
    )(p, p, c, r1, w1lT, w2rT, b2l2d)


def _tc3(p, c, r2, w2lT, wc1T, bc12d, wc2Tp, bc2p):
    return pl.pallas_call(
        _tc3_body,
        grid=(NPAD // BLK,),
        in_specs=[_part_spec(D, 0), _part_spec(D, 1), _cnt_spec(),
                  _rows_spec(D), _full_spec(D, D), _full_spec(D, FC),
                  _full_spec(1, FC), _full_spec(FC, D), _full_spec(1, D)],
        out_specs=[_rows_spec(D), _rows_spec(D)],
        out_shape=[jax.ShapeDtypeStruct((NPAD, D), jnp.float32),
                   jax.ShapeDtypeStruct((NPAD, D), jnp.float32)],
    )(p, p, c, r2, w2lT, wc1T, bc12d, wc2Tp, bc2p)


def kernel(x, edge_index, W1l, b1l, W1r, W2l, b2l, W2r, Wc1, bc1, Wc2, bc2):
    xp = jnp.pad(x, ((0, NPAD - N), (0, 0)))
    src = edge_index[0]
    dst = edge_index[1]
    srcp = jnp.pad(src, (0, EPAD - E)).reshape(EROWS, CHUNK)
    dstp = jnp.pad(dst, (0, EPAD - E), constant_values=N).reshape(EROWS, CHUNK)
    zero_d = jnp.zeros((NPAD, D), jnp.float32)

    p1, cnt32 = _seg_sum_sc(xp, srcp, dstp, zero_d, True)
    r1 = _tc1(xp, W1r.T, b1l[None, :])
    h, r2 = _tc2(p1, cnt32, r1, W1l.T, W2r.T, b2l[None, :])
    (p2,) = _seg_sum_sc(h, srcp, dstp, zero_d, False)
    wc2Tp = jnp.pad(Wc2.T, ((0, 0), (0, D - NC)))
    bc2p = jnp.pad(bc2, (0, D - NC))[None, :]
    h2p, outp = _tc3(p2, cnt32, r2, W2l.T, Wc1.T, bc1[None, :], wc2Tp, bc2p)

    out = outp[:N, :NC]
    h2 = h2p[:N]
    node_mask = (jax.random.uniform(jax.random.key(1), (N, 1)) > 0.1)
    return (out, node_mask.astype(jnp.float32), h2)
